# baseline (speedup 1.0000x reference)
import functools

import jax
import jax.numpy as jnp
from jax import lax
from jax.experimental import pallas as pl
from jax.experimental.pallas import tpu as pltpu
from jax.experimental.pallas import tpu_sc as plsc

_N = 100000
_NW = 32
_LPW = 3128
_G = 196
_ROWS = 2 * _LPW


def _sc_body(q_hbm, pt_hbm, outb_hbm, outa_hbm,
             buf0, buf1, buf2, buf3, pbuf, obv, oav,
             sem0, sem1, sem2, sem3):
    wid = lax.axis_index("s") * 2 + lax.axis_index("c")
    base = jnp.minimum(wid * _LPW, _N - _LPW)
    qbase = 2 * base
    pltpu.sync_copy(pt_hbm.at[pl.ds(2 * base, 2 * _LPW)], pbuf)

    iota = lax.broadcasted_iota(jnp.int32, (16,), 0)
    zeros = jnp.zeros((16,), jnp.int32)
    ones = zeros + 1

    def rb(g):
        return jnp.minimum(g * 32, _ROWS - 32)

    def start(g, buf, sem):
        pltpu.make_async_copy(
            q_hbm.at[pl.ds(qbase + rb(g), 32)],
            buf.at[:, pl.ds(0, 128)], sem).start()

    def wait(g, buf, sem):
        pltpu.make_async_copy(
            q_hbm.at[pl.ds(qbase + rb(g), 32)],
            buf.at[:, pl.ds(0, 128)], sem).wait()

    bufs = (buf0, buf1, buf2, buf3)
    sems = (sem0, sem1, sem2, sem3)
    for b in range(4):
        start(b, bufs[b], sems[b])

    def outer(o, carry):
        for b in range(4):
            g = o * 4 + b
            buf = bufs[b]
            sem = sems[b]
            wait(g, buf, sem)
            tl = jnp.minimum(g * 16 + iota, _LPW - 1)
            rowx = 2 * tl - rb(g)
            rowy = rowx + 1
            px = plsc.load_gather(pbuf, [2 * tl])
            py = plsc.load_gather(pbuf, [2 * tl + 1])

            def inner(i, c):
                dmin, imin, ci = c
                x = plsc.load_gather(buf, [rowx, ci])
                y = plsc.load_gather(buf, [rowy, ci])
                dx = x - px
                dy = y - py
                dd = dx * dx + dy * dy
                bt = dd < dmin
                return (jnp.where(bt, dd, dmin),
                        jnp.where(bt, ci, imin),
                        ci + 1)

            dmin0 = jnp.full((16,), jnp.inf, jnp.float32)
            dmin, imin, _ = lax.fori_loop(
                1, 127, inner, (dmin0, zeros, ones), unroll=7)

            cm = imin
            xc = plsc.load_gather(buf, [rowx, cm])
            yc = plsc.load_gather(buf, [rowy, cm])
            xp = plsc.load_gather(buf, [rowx, cm - 1])
            yp = plsc.load_gather(buf, [rowy, cm - 1])
            xn = plsc.load_gather(buf, [rowx, cm + 1])
            yn = plsc.load_gather(buf, [rowy, cm + 1])
            dxp = xp - xc
            dyp = yp - yc
            dxn = xn - xc
            dyn = yn - yc
            dp = dxp * dxp + dyp * dyp
            dn = dxn * dxn + dyn * dyn
            bef = cm - jnp.where(dn < dp, 0, 1)
            plsc.store_scatter(obv, [tl], bef)
            plsc.store_scatter(oav, [tl], bef + 1)

            @pl.when(g + 4 < _G)
            def _():
                start(g + 4, buf, sem)
        return carry

    lax.fori_loop(0, _G // 4, outer, 0)

    pltpu.sync_copy(obv, outb_hbm.at[pl.ds(base, _LPW)])
    pltpu.sync_copy(oav, outa_hbm.at[pl.ds(base, _LPW)])


@jax.jit
def _run_sc(q, pt):
    f = pl.kernel(
        _sc_body,
        out_type=[
            jax.ShapeDtypeStruct((_N,), jnp.int32),
            jax.ShapeDtypeStruct((_N,), jnp.int32),
        ],
        mesh=plsc.VectorSubcoreMesh(core_axis_name="c", subcore_axis_name="s"),
        compiler_params=pltpu.CompilerParams(needs_layout_passes=False),
        scratch_types=[
            pltpu.VMEM((32, 132), jnp.float32),
            pltpu.VMEM((32, 132), jnp.float32),
            pltpu.VMEM((32, 132), jnp.float32),
            pltpu.VMEM((32, 132), jnp.float32),
            pltpu.VMEM((2 * _LPW,), jnp.float32),
            pltpu.VMEM((_LPW,), jnp.int32),
            pltpu.VMEM((_LPW,), jnp.int32),
            pltpu.SemaphoreType.DMA,
            pltpu.SemaphoreType.DMA,
            pltpu.SemaphoreType.DMA,
            pltpu.SemaphoreType.DMA,
        ],
    )
    return f(q, pt)


def kernel(line_nodes, point):
    n = point.shape[0]
    q = line_nodes.transpose(0, 2, 1).reshape(2 * n, 128)
    ob, oa = _run_sc(q, point.reshape(2 * n))
    return ob, oa


# speedup vs baseline: 1.0865x; 1.0865x over previous
import functools

import jax
import jax.numpy as jnp
from jax import lax
from jax.experimental import pallas as pl
from jax.experimental.pallas import tpu as pltpu
from jax.experimental.pallas import tpu_sc as plsc

_N = 100000
_NW = 32
_LPW = 3128
_G = 196
_ROWS = 2 * _LPW


def _sc_body(q_hbm, pt_hbm, outb_hbm, outa_hbm,
             buf0, buf1, buf2, buf3, pbuf, obv, oav,
             sem0, sem1, sem2, sem3):
    wid = lax.axis_index("s") * 2 + lax.axis_index("c")
    base = jnp.minimum(wid * _LPW, _N - _LPW)
    qbase = 2 * base
    pltpu.sync_copy(pt_hbm.at[pl.ds(2 * base, 2 * _LPW)], pbuf)

    iota = lax.broadcasted_iota(jnp.int32, (16,), 0)
    zeros = jnp.zeros((16,), jnp.int32)
    ones = zeros + 1

    def rb(g):
        return jnp.minimum(g * 32, _ROWS - 32)

    def start(g, buf, sem):
        pltpu.make_async_copy(
            q_hbm.at[pl.ds(qbase + rb(g), 32)],
            buf, sem).start()

    def wait(g, buf, sem):
        pltpu.make_async_copy(
            q_hbm.at[pl.ds(qbase + rb(g), 32)],
            buf, sem).wait()

    bufs = (buf0, buf1, buf2, buf3)
    sems = (sem0, sem1, sem2, sem3)
    for b in range(4):
        start(b, bufs[b], sems[b])

    def outer(o, carry):
        for b in range(4):
            g = o * 4 + b
            buf = bufs[b]
            sem = sems[b]
            wait(g, buf, sem)
            tl = jnp.minimum(g * 16 + iota, _LPW - 1)
            rowx = 2 * tl - rb(g)
            rowy = rowx + 1
            px = plsc.load_gather(pbuf, [2 * tl])
            py = plsc.load_gather(pbuf, [2 * tl + 1])

            def inner(i, c):
                dmin, imin, ci = c
                x = plsc.load_gather(buf, [rowx, ci])
                y = plsc.load_gather(buf, [rowy, ci])
                dx = x - px
                dy = y - py
                dd = dx * dx + dy * dy
                bt = dd < dmin
                return (jnp.where(bt, dd, dmin),
                        jnp.where(bt, ci, imin),
                        ci + 1)

            dmin0 = jnp.full((16,), jnp.inf, jnp.float32)
            dmin, imin, _ = lax.fori_loop(
                1, 127, inner, (dmin0, zeros, ones), unroll=18)

            cm = imin
            xc = plsc.load_gather(buf, [rowx, cm])
            yc = plsc.load_gather(buf, [rowy, cm])
            xp = plsc.load_gather(buf, [rowx, cm - 1])
            yp = plsc.load_gather(buf, [rowy, cm - 1])
            xn = plsc.load_gather(buf, [rowx, cm + 1])
            yn = plsc.load_gather(buf, [rowy, cm + 1])
            dxp = xp - xc
            dyp = yp - yc
            dxn = xn - xc
            dyn = yn - yc
            dp = dxp * dxp + dyp * dyp
            dn = dxn * dxn + dyn * dyn
            bef = cm - jnp.where(dn < dp, 0, 1)
            plsc.store_scatter(obv, [tl], bef)
            plsc.store_scatter(oav, [tl], bef + 1)

            @pl.when(g + 4 < _G)
            def _():
                start(g + 4, buf, sem)
        return carry

    lax.fori_loop(0, _G // 4, outer, 0)

    pltpu.sync_copy(obv, outb_hbm.at[pl.ds(base, _LPW)])
    pltpu.sync_copy(oav, outa_hbm.at[pl.ds(base, _LPW)])


@jax.jit
def _run_sc(q, pt):
    f = pl.kernel(
        _sc_body,
        out_type=[
            jax.ShapeDtypeStruct((_N,), jnp.int32),
            jax.ShapeDtypeStruct((_N,), jnp.int32),
        ],
        mesh=plsc.VectorSubcoreMesh(core_axis_name="c", subcore_axis_name="s"),
        compiler_params=pltpu.CompilerParams(needs_layout_passes=False),
        scratch_types=[
            pltpu.VMEM((32, 128), jnp.float32),
            pltpu.VMEM((32, 128), jnp.float32),
            pltpu.VMEM((32, 128), jnp.float32),
            pltpu.VMEM((32, 128), jnp.float32),
            pltpu.VMEM((2 * _LPW,), jnp.float32),
            pltpu.VMEM((_LPW,), jnp.int32),
            pltpu.VMEM((_LPW,), jnp.int32),
            pltpu.SemaphoreType.DMA,
            pltpu.SemaphoreType.DMA,
            pltpu.SemaphoreType.DMA,
            pltpu.SemaphoreType.DMA,
        ],
    )
    return f(q, pt)


def kernel(line_nodes, point):
    n = point.shape[0]
    q = line_nodes.transpose(0, 2, 1).reshape(2 * n, 128)
    ob, oa = _run_sc(q, point.reshape(2 * n))
    return ob, oa


# speedup vs baseline: 1.4961x; 1.3770x over previous
import jax
import jax.numpy as jnp
import numpy as np
from jax import lax
from jax.experimental import pallas as pl
from jax.experimental.pallas import tpu as pltpu
from jax.experimental.pallas import tpu_sc as plsc

_N = 100000
_KTC = 49824
_B = 2076
_NW = 32
_LPW = 1568
_G = 98
_ROWS = 2 * _LPW


def _tc_body(w_ref, pt_ref, pen_ref, o_ref):
    w = w_ref[...]
    p = pt_ref[...]
    rows = 2 * _B

    lanes = jax.lax.broadcasted_iota(jnp.int32, w.shape, 1)
    srows = jax.lax.broadcasted_iota(jnp.int32, (rows, 1), 0)
    even_s = (srows & 1) == 0

    df = w - p
    sq = df * df
    dm = sq + pltpu.roll(sq, rows - 1, 0) + pen_ref[...]

    mval = jnp.min(dm, axis=1, keepdims=True)
    eq = dm == mval
    minlane = jnp.min(jnp.where(eq, lanes, 127), axis=1, keepdims=True)

    g = pltpu.roll(w, 127, 1) - w
    gsq = g * g
    u = gsq + pltpu.roll(gsq, rows - 1, 0)

    first = lanes == minlane
    ddiff = jnp.sum(jnp.where(first, u - pltpu.roll(u, 1, 1), 0.0),
                    axis=1, keepdims=True)

    before = minlane - (ddiff >= 0.0).astype(jnp.int32)
    o_ref[...] = jnp.where(even_s, before, pltpu.roll(before, 1, 0) + 1)


_PEN = np.zeros((1, 128), dtype=np.float32)
_PEN[0, 0] = np.inf
_PEN[0, 127] = np.inf


def _sc_body(q_hbm, pt_hbm, outb_hbm, outa_hbm,
             buf0, buf1, pbuf, obv, oav, sem0, sem1):
    wid = lax.axis_index("s") * 2 + lax.axis_index("c")
    base = _KTC + wid * _LPW
    qbase = 2 * base
    pltpu.sync_copy(pt_hbm.at[pl.ds(2 * base, 2 * _LPW)], pbuf)

    iota = lax.broadcasted_iota(jnp.int32, (16,), 0)
    zeros = jnp.zeros((16,), jnp.int32)
    ones = zeros + 1

    def start(g, buf, sem):
        pltpu.make_async_copy(
            q_hbm.at[pl.ds(qbase + g * 32, 32)], buf, sem).start()

    def wait(g, buf, sem):
        pltpu.make_async_copy(
            q_hbm.at[pl.ds(qbase + g * 32, 32)], buf, sem).wait()

    bufs = (buf0, buf1)
    sems = (sem0, sem1)
    start(0, buf0, sem0)
    start(1, buf1, sem1)

    def outer(o, carry):
        for b in range(2):
            g = o * 2 + b
            buf = bufs[b]
            sem = sems[b]
            wait(g, buf, sem)
            tl = g * 16 + iota
            rowx = 2 * iota
            rowy = rowx + 1
            px = plsc.load_gather(pbuf, [2 * tl])
            py = plsc.load_gather(pbuf, [2 * tl + 1])

            def inner(i, c):
                dmin, imin, ci = c
                x = plsc.load_gather(buf, [rowx, ci])
                y = plsc.load_gather(buf, [rowy, ci])
                dx = x - px
                dy = y - py
                dd = dx * dx + dy * dy
                bt = dd < dmin
                return (jnp.where(bt, dd, dmin),
                        jnp.where(bt, ci, imin),
                        ci + 1)

            dmin0 = jnp.full((16,), jnp.inf, jnp.float32)
            dmin, imin, _ = lax.fori_loop(
                1, 127, inner, (dmin0, zeros, ones), unroll=18)

            cm = imin
            xc = plsc.load_gather(buf, [rowx, cm])
            yc = plsc.load_gather(buf, [rowy, cm])
            xp = plsc.load_gather(buf, [rowx, cm - 1])
            yp = plsc.load_gather(buf, [rowy, cm - 1])
            xn = plsc.load_gather(buf, [rowx, cm + 1])
            yn = plsc.load_gather(buf, [rowy, cm + 1])
            dxp = xp - xc
            dyp = yp - yc
            dxn = xn - xc
            dyn = yn - yc
            dp = dxp * dxp + dyp * dyp
            dn = dxn * dxn + dyn * dyn
            bef = cm - jnp.where(dn < dp, 0, 1)
            plsc.store_scatter(obv, [tl], bef)
            plsc.store_scatter(oav, [tl], bef + 1)

            @pl.when(g + 2 < _G)
            def _():
                start(g + 2, buf, sem)
        return carry

    lax.fori_loop(0, _G // 2, outer, 0)

    pltpu.sync_copy(obv, outb_hbm.at[pl.ds(wid * _LPW, _LPW)])
    pltpu.sync_copy(oav, outa_hbm.at[pl.ds(wid * _LPW, _LPW)])


@jax.jit
def _run(q, p2, pen):
    sc = pl.kernel(
        _sc_body,
        out_type=[
            jax.ShapeDtypeStruct((_NW * _LPW,), jnp.int32),
            jax.ShapeDtypeStruct((_NW * _LPW,), jnp.int32),
        ],
        mesh=plsc.VectorSubcoreMesh(core_axis_name="c", subcore_axis_name="s"),
        compiler_params=pltpu.CompilerParams(needs_layout_passes=False),
        scratch_types=[
            pltpu.VMEM((32, 128), jnp.float32),
            pltpu.VMEM((32, 128), jnp.float32),
            pltpu.VMEM((2 * _LPW,), jnp.float32),
            pltpu.VMEM((_LPW,), jnp.int32),
            pltpu.VMEM((_LPW,), jnp.int32),
            pltpu.SemaphoreType.DMA,
            pltpu.SemaphoreType.DMA,
        ],
    )
    scb, sca = sc(q, p2.reshape(-1))

    o = pl.pallas_call(
        _tc_body,
        grid=(_KTC // _B,),
        in_specs=[
            pl.BlockSpec((2 * _B, 128), lambda i: (i, 0)),
            pl.BlockSpec((2 * _B, 1), lambda i: (i, 0)),
            pl.BlockSpec((1, 128), lambda i: (0, 0)),
        ],
        out_specs=pl.BlockSpec((2 * _B, 1), lambda i: (i, 0)),
        out_shape=jax.ShapeDtypeStruct((2 * _KTC, 1), jnp.int32),
        compiler_params=pltpu.CompilerParams(
            dimension_semantics=("arbitrary",),
        ),
    )(q, p2, pen)
    r = o.reshape(_KTC, 2)
    before = jnp.concatenate([r[:, 0], scb])
    after = jnp.concatenate([r[:, 1], sca])
    return before, after


def kernel(line_nodes, point):
    n = point.shape[0]
    q = line_nodes.transpose(0, 2, 1).reshape(2 * n, 128)
    p2 = point.reshape(2 * n, 1)
    return _run(q, p2, jnp.asarray(_PEN))


# speedup vs baseline: 1.5555x; 1.0397x over previous
import jax
import jax.numpy as jnp
import numpy as np
from jax import lax
from jax.experimental import pallas as pl
from jax.experimental.pallas import tpu as pltpu
from jax.experimental.pallas import tpu_sc as plsc

_N = 100000
_KTC = 44192
_B = 5524
_NW = 32
_LPW = 1744
_G = 110
_ROWS = 2 * _LPW


def _tc_body(w_ref, pt_ref, pen_ref, o_ref):
    w = w_ref[...]
    p = pt_ref[...]
    rows = 2 * _B

    lanes = jax.lax.broadcasted_iota(jnp.int32, w.shape, 1)
    srows = jax.lax.broadcasted_iota(jnp.int32, (rows, 1), 0)
    even_s = (srows & 1) == 0

    df = w - p
    sq = df * df
    dm = sq + pltpu.roll(sq, rows - 1, 0) + pen_ref[...]

    mval = jnp.min(dm, axis=1, keepdims=True)
    eq = dm == mval
    minlane = jnp.min(jnp.where(eq, lanes, 127), axis=1, keepdims=True)

    g = pltpu.roll(w, 127, 1) - w
    gsq = g * g
    u = gsq + pltpu.roll(gsq, rows - 1, 0)

    first = lanes == minlane
    ddiff = jnp.sum(jnp.where(first, u - pltpu.roll(u, 1, 1), 0.0),
                    axis=1, keepdims=True)

    before = minlane - (ddiff >= 0.0).astype(jnp.int32)
    o_ref[...] = jnp.where(even_s, before, pltpu.roll(before, 1, 0) + 1)


_PEN = np.zeros((1, 128), dtype=np.float32)
_PEN[0, 0] = np.inf
_PEN[0, 127] = np.inf


def _sc_body(q_hbm, pt_hbm, outb_hbm, outa_hbm,
             buf0, buf1, pbuf, obv, oav, sem0, sem1):
    wid = lax.axis_index("s") * 2 + lax.axis_index("c")
    base = _KTC + wid * _LPW
    qbase = 2 * base
    pltpu.sync_copy(pt_hbm.at[pl.ds(2 * base, 2 * _LPW)], pbuf)

    iota = lax.broadcasted_iota(jnp.int32, (16,), 0)
    zeros = jnp.zeros((16,), jnp.int32)
    ones = zeros + 1

    def rb(g):
        return jnp.minimum(g * 32, _ROWS - 32)

    def start(g, buf, sem):
        pltpu.make_async_copy(
            q_hbm.at[pl.ds(qbase + rb(g), 32)], buf, sem).start()

    def wait(g, buf, sem):
        pltpu.make_async_copy(
            q_hbm.at[pl.ds(qbase + rb(g), 32)], buf, sem).wait()

    bufs = (buf0, buf1)
    sems = (sem0, sem1)
    start(0, buf0, sem0)
    start(1, buf1, sem1)

    def outer(o, carry):
        for b in range(2):
            g = o * 2 + b
            buf = bufs[b]
            sem = sems[b]
            wait(g, buf, sem)
            tl = jnp.minimum(g * 16 + iota, _LPW - 1)
            rowx = 2 * tl - rb(g)
            rowy = rowx + 1
            px = plsc.load_gather(pbuf, [2 * tl])
            py = plsc.load_gather(pbuf, [2 * tl + 1])

            def inner(i, c):
                dmin, imin, ci = c
                x = plsc.load_gather(buf, [rowx, ci])
                y = plsc.load_gather(buf, [rowy, ci])
                dx = x - px
                dy = y - py
                dd = dx * dx + dy * dy
                bt = dd < dmin
                return (jnp.where(bt, dd, dmin),
                        jnp.where(bt, ci, imin),
                        ci + 1)

            dmin0 = jnp.full((16,), jnp.inf, jnp.float32)
            dmin, imin, _ = lax.fori_loop(
                1, 127, inner, (dmin0, zeros, ones), unroll=18)

            cm = imin
            xc = plsc.load_gather(buf, [rowx, cm])
            yc = plsc.load_gather(buf, [rowy, cm])
            xp = plsc.load_gather(buf, [rowx, cm - 1])
            yp = plsc.load_gather(buf, [rowy, cm - 1])
            xn = plsc.load_gather(buf, [rowx, cm + 1])
            yn = plsc.load_gather(buf, [rowy, cm + 1])
            dxp = xp - xc
            dyp = yp - yc
            dxn = xn - xc
            dyn = yn - yc
            dp = dxp * dxp + dyp * dyp
            dn = dxn * dxn + dyn * dyn
            bef = cm - jnp.where(dn < dp, 0, 1)
            plsc.store_scatter(obv, [tl], bef)
            plsc.store_scatter(oav, [tl], bef + 1)

            @pl.when(g + 2 < _G)
            def _():
                start(g + 2, buf, sem)
        return carry

    lax.fori_loop(0, _G // 2, outer, 0)

    pltpu.sync_copy(obv, outb_hbm.at[pl.ds(wid * _LPW, _LPW)])
    pltpu.sync_copy(oav, outa_hbm.at[pl.ds(wid * _LPW, _LPW)])


@jax.jit
def _run(q, p2, pen):
    sc = pl.kernel(
        _sc_body,
        out_type=[
            jax.ShapeDtypeStruct((_NW * _LPW,), jnp.int32),
            jax.ShapeDtypeStruct((_NW * _LPW,), jnp.int32),
        ],
        mesh=plsc.VectorSubcoreMesh(core_axis_name="c", subcore_axis_name="s"),
        compiler_params=pltpu.CompilerParams(needs_layout_passes=False),
        scratch_types=[
            pltpu.VMEM((32, 128), jnp.float32),
            pltpu.VMEM((32, 128), jnp.float32),
            pltpu.VMEM((2 * _LPW,), jnp.float32),
            pltpu.VMEM((_LPW,), jnp.int32),
            pltpu.VMEM((_LPW,), jnp.int32),
            pltpu.SemaphoreType.DMA,
            pltpu.SemaphoreType.DMA,
        ],
    )
    scb, sca = sc(q, p2.reshape(-1))

    o = pl.pallas_call(
        _tc_body,
        grid=(_KTC // _B,),
        in_specs=[
            pl.BlockSpec((2 * _B, 128), lambda i: (i, 0)),
            pl.BlockSpec((2 * _B, 1), lambda i: (i, 0)),
            pl.BlockSpec((1, 128), lambda i: (0, 0)),
        ],
        out_specs=pl.BlockSpec((2 * _B, 1), lambda i: (i, 0)),
        out_shape=jax.ShapeDtypeStruct((2 * _KTC, 1), jnp.int32),
        compiler_params=pltpu.CompilerParams(
            dimension_semantics=("arbitrary",),
        ),
    )(q, p2, pen)
    r = o.reshape(_KTC, 2)
    before = jnp.concatenate([r[:, 0], scb])
    after = jnp.concatenate([r[:, 1], sca])
    return before, after


def kernel(line_nodes, point):
    n = point.shape[0]
    q = line_nodes.transpose(0, 2, 1).reshape(2 * n, 128)
    p2 = point.reshape(2 * n, 1)
    return _run(q, p2, jnp.asarray(_PEN))
